# Initial kernel scaffold; baseline (speedup 1.0000x reference)
import jax
import jax.numpy as jnp
from jax.experimental import pallas as pl


def kernel(x, edge_index, gaussian_noise, params):
    raise NotImplementedError("write your pallas kernel here")


# speedup vs baseline: 4.0403x; 4.0403x over previous
import functools

import jax
import jax.numpy as jnp
from jax import lax
from jax.experimental import pallas as pl
from jax.experimental.pallas import tpu as pltpu
from jax.experimental.pallas import tpu_sc as plsc

NC = 2
NS = 16
NW = NC * NS
CHUNK = 128

ROW_BLK = 2000


def _make_segsum(n_rows, h, k_chunks, acc_rows):
    mesh = plsc.VectorSubcoreMesh(core_axis_name="c", subcore_axis_name="s")
    rpt = acc_rows // NS

    @functools.partial(
        pl.kernel,
        mesh=mesh,
        out_type=jax.ShapeDtypeStruct((NC, acc_rows, h), jnp.float32),
        scratch_types=[
            pltpu.VMEM((k_chunks, CHUNK), jnp.int32),
            pltpu.VMEM((k_chunks, CHUNK), jnp.int32),
            pltpu.VMEM((CHUNK, h), jnp.float32),
            pltpu.VMEM_SHARED((acc_rows, h), jnp.float32),
            pltpu.SemaphoreType.DMA,
        ],
    )
    def segsum(table_hbm, src_hbm, dst_hbm, zeros_hbm, out_hbm,
               src_v, dst_v, rows_v, acc, sem):
        c = lax.axis_index("c")
        s = lax.axis_index("s")
        wid = s * NC + c
        pltpu.sync_copy(zeros_hbm.at[pl.ds(s * rpt, rpt)],
                        acc.at[pl.ds(s * rpt, rpt)])
        pltpu.sync_copy(src_hbm.at[wid], src_v)
        pltpu.sync_copy(dst_hbm.at[wid], dst_v)
        plsc.subcore_barrier()

        def body(j, carry):
            pltpu.async_copy(table_hbm.at[src_v.at[j]], rows_v, sem).wait()
            pltpu.sync_copy(rows_v, acc.at[dst_v.at[j]], add=True)
            return carry

        lax.fori_loop(0, k_chunks, body, 0)
        plsc.subcore_barrier()
        pltpu.sync_copy(acc.at[pl.ds(s * rpt, rpt)],
                        out_hbm.at[c, pl.ds(s * rpt, rpt)])

    return segsum


def _mlp1_body(x_ref, a0_ref, a1_ref, w_ref, b_ref, t_ref, stats_ref):
    hcols = t_ref.shape[1]
    hid = x_ref[...] + a0_ref[...] + a1_ref[...]
    t = jnp.dot(hid, w_ref[...], preferred_element_type=jnp.float32) + b_ref[...]
    t_ref[...] = t

    @pl.when(pl.program_id(0) == 0)
    def _():
        stats_ref[...] = jnp.zeros_like(stats_ref)

    sums = jnp.concatenate(
        [jnp.sum(t, axis=0, keepdims=True),
         jnp.sum(t * t, axis=0, keepdims=True),
         jnp.zeros((6, hcols), jnp.float32)],
        axis=0,
    )
    stats_ref[...] += sums


def _mlp1(x, a0, a1, w1, b1, n_rows):
    h = x.shape[1]
    h2 = w1.shape[1]
    grid = n_rows // ROW_BLK
    return pl.pallas_call(
        _mlp1_body,
        grid=(grid,),
        in_specs=[
            pl.BlockSpec((ROW_BLK, h), lambda i: (i, 0)),
            pl.BlockSpec((ROW_BLK, h), lambda i: (i, 0)),
            pl.BlockSpec((ROW_BLK, h), lambda i: (i, 0)),
            pl.BlockSpec((h, h2), lambda i: (0, 0)),
            pl.BlockSpec((1, h2), lambda i: (0, 0)),
        ],
        out_specs=[
            pl.BlockSpec((ROW_BLK, h2), lambda i: (i, 0)),
            pl.BlockSpec((8, h2), lambda i: (0, 0)),
        ],
        out_shape=[
            jax.ShapeDtypeStruct((n_rows, h2), jnp.float32),
            jax.ShapeDtypeStruct((8, h2), jnp.float32),
        ],
    )(x, a0, a1, w1, b1.reshape(1, h2))


def _mlp2_body(t_ref, stats_ref, g_ref, be_ref, w_ref, b_ref, o_ref,
               *, relu, n_rows, final):
    inv_n = 1.0 / n_rows
    m = stats_ref[0:1, :] * inv_n
    v = stats_ref[1:2, :] * inv_n - m * m
    scale = lax.rsqrt(v + 1e-5) * g_ref[...]
    hid = (t_ref[...] - m) * scale + be_ref[...]
    if relu:
        hid = jnp.maximum(hid, 0.0)
    o = jnp.dot(hid, w_ref[...], preferred_element_type=jnp.float32) + b_ref[...]
    o_ref[...] = o


def _mlp2_final_body(t_ref, stats_ref, g_ref, be_ref, w_ref, b_ref,
                     mean_ref, noise_ref, o_ref, *, n_rows):
    inv_n = 1.0 / n_rows
    m = stats_ref[0:1, :] * inv_n
    v = stats_ref[1:2, :] * inv_n - m * m
    scale = lax.rsqrt(v + 1e-5) * g_ref[...]
    hid = (t_ref[...] - m) * scale + be_ref[...]
    o = jnp.dot(hid, w_ref[...], preferred_element_type=jnp.float32) + b_ref[...]
    o_ref[...] = noise_ref[...] * jnp.exp(o) + mean_ref[...]


def _mlp2(t, stats, g, be, w2, b2, relu, n_rows, mean=None, noise=None):
    h2 = t.shape[1]
    h = w2.shape[1]
    grid = n_rows // ROW_BLK
    in_specs = [
        pl.BlockSpec((ROW_BLK, h2), lambda i: (i, 0)),
        pl.BlockSpec((8, h2), lambda i: (0, 0)),
        pl.BlockSpec((1, h2), lambda i: (0, 0)),
        pl.BlockSpec((1, h2), lambda i: (0, 0)),
        pl.BlockSpec((h2, h), lambda i: (0, 0)),
        pl.BlockSpec((1, h), lambda i: (0, 0)),
    ]
    args = [t, stats, g.reshape(1, h2), be.reshape(1, h2), w2, b2.reshape(1, h)]
    if mean is None:
        body = functools.partial(_mlp2_body, relu=relu, n_rows=n_rows, final=False)
    else:
        body = functools.partial(_mlp2_final_body, n_rows=n_rows)
        in_specs += [
            pl.BlockSpec((ROW_BLK, h), lambda i: (i, 0)),
            pl.BlockSpec((ROW_BLK, h), lambda i: (i, 0)),
        ]
        args += [mean, noise]
    return pl.pallas_call(
        body,
        grid=(grid,),
        in_specs=in_specs,
        out_specs=pl.BlockSpec((ROW_BLK, h), lambda i: (i, 0)),
        out_shape=jax.ShapeDtypeStruct((n_rows, h), jnp.float32),
    )(*args)


def kernel(x, edge_index, gaussian_noise, params):
    n, h = x.shape
    e = edge_index.shape[1]
    acc_rows = ((n + NS * 8) // (NS * 8)) * (NS * 8)

    epw = -(-e // NW)
    k_chunks = -(-epw // CHUNK)
    e_pad = NW * k_chunks * CHUNK
    src = jnp.concatenate(
        [edge_index[0], jnp.zeros((e_pad - e,), jnp.int32)]).reshape(NW, k_chunks, CHUNK)
    dst = jnp.concatenate(
        [edge_index[1], jnp.full((e_pad - e,), n, jnp.int32)]).reshape(NW, k_chunks, CHUNK)
    zeros = jnp.zeros((acc_rows, h), jnp.float32)

    segsum = _make_segsum(n, h, k_chunks, acc_rows)

    def gin_dense(h_in, parts, p, relu):
        t, stats = _mlp1(h_in, parts[0, :n], parts[1, :n], p["W1"], p["b1"], n)
        return _mlp2(t, stats, p["g"], p["be"], p["W2"], p["b2"], relu, n)

    p0 = segsum(x, src, dst, zeros)
    h0 = gin_dense(x, p0, params["c0"], True)
    p1 = segsum(h0, src, dst, zeros)
    h1 = gin_dense(h0, p1, params["c1"], True)
    p2 = segsum(h1, src, dst, zeros)
    mean = gin_dense(h1, p2, params["c2"], False)
    p3 = params["c3"]
    t3, st3 = _mlp1(h1, p2[0, :n], p2[1, :n], p3["W1"], p3["b1"], n)
    z = _mlp2(t3, st3, p3["g"], p3["be"], p3["W2"], p3["b2"], False, n,
              mean=mean, noise=gaussian_noise)
    return z
